# baseline (speedup 1.0000x reference)
import functools
import numpy as np
import jax
import jax.numpy as jnp
from jax import lax
from jax.experimental import pallas as pl
from jax.experimental.pallas import tpu as pltpu

_F32 = jnp.float32


def _pcall(body, **kw):
    return pl.pallas_call(body, **kw)


def _conv2d(x, w, groups=1):
    return lax.conv_general_dilated(
        x, w, (1, 1), 'SAME',
        dimension_numbers=('NCHW', 'OIHW', 'NCHW'),
        feature_group_count=groups)


def _pixel_unshuffle(x, r):
    B, C, H, W = x.shape
    x = x.reshape(B, C, H // r, r, W // r, r)
    return x.transpose(0, 1, 3, 5, 2, 4).reshape(B, C * r * r, H // r, W // r)


def _pixel_shuffle(x, r):
    B, C, H, W = x.shape
    x = x.reshape(B, C // (r * r), r, r, H, W)
    return x.transpose(0, 1, 4, 2, 5, 3).reshape(B, C // (r * r), H * r, W * r)


def _win(t, n_win, h, w, C):
    return (t.reshape(n_win, h, n_win, w, C)
             .transpose(0, 2, 1, 3, 4)
             .reshape(n_win * n_win, h * w, C))


def _unwin(t, n_win, h, w, C):
    return (t.reshape(n_win, n_win, h, w, C)
             .transpose(0, 2, 1, 3, 4)
             .reshape(n_win * h, n_win * w, C))


@functools.lru_cache(maxsize=None)
def _pool_matrix(h, w, kv):
    m = kv * kv
    bh, bw = h // kv, w // kv
    pm = np.zeros((m + 1, h * w), np.float32)
    for ki in range(kv):
        for kj in range(kv):
            for r in range(bh):
                for c in range(bw):
                    pm[ki * kv + kj, (ki * bh + r) * w + (kj * bw + c)] = 1.0 / (bh * bw)
    pm[m, :] = 1.0 / (h * w)
    return jnp.asarray(pm)


def _stage_a(xw, lng, lnb, wq, wk, wv, bq, bk, bv, pmat):
    P, hw, C = xw.shape
    m1 = pmat.shape[0]
    m = m1 - 1

    def body(xw_ref, g_ref, b_ref, wq_ref, wk_ref, wv_ref,
             bq_ref, bk_ref, bv_ref, pm_ref,
             qw_ref, vw_ref, kpm_ref, vp_ref, qm_ref):
        x = xw_ref[0]
        mu = jnp.mean(x, axis=-1, keepdims=True)
        xc = x - mu
        var = jnp.mean(xc * xc, axis=-1, keepdims=True)
        xn = xc * lax.rsqrt(var + 1e-5) * g_ref[...] + b_ref[...]
        q = jnp.dot(xn, wq_ref[...], preferred_element_type=_F32) + bq_ref[...]
        k = jnp.dot(xn, wk_ref[...], preferred_element_type=_F32) + bk_ref[...]
        v = jnp.dot(xn, wv_ref[...], preferred_element_type=_F32) + bv_ref[...]
        qw_ref[0] = q
        vw_ref[0] = v
        pm = pm_ref[...]
        kpm_ref[0] = jnp.dot(pm, k, preferred_element_type=_F32)
        vp_ref[0] = jnp.dot(pm[:m], v, preferred_element_type=_F32)
        qm_ref[0] = jnp.dot(pm[m:m1], q, preferred_element_type=_F32)

    full = lambda shape: pl.BlockSpec(shape, lambda i: (0,) * len(shape))
    out = _pcall(
        body,
        grid=(P,),
        in_specs=[pl.BlockSpec((1, hw, C), lambda i: (i, 0, 0)),
                  full((1, C)), full((1, C)),
                  full((C, C)), full((C, C)), full((C, C)),
                  full((1, C)), full((1, C)), full((1, C)),
                  full((m1, hw))],
        out_specs=[pl.BlockSpec((1, hw, C), lambda i: (i, 0, 0)),
                   pl.BlockSpec((1, hw, C), lambda i: (i, 0, 0)),
                   pl.BlockSpec((1, m1, C), lambda i: (i, 0, 0)),
                   pl.BlockSpec((1, m, C), lambda i: (i, 0, 0)),
                   pl.BlockSpec((1, 1, C), lambda i: (i, 0, 0))],
        out_shape=[jax.ShapeDtypeStruct((P, hw, C), _F32),
                   jax.ShapeDtypeStruct((P, hw, C), _F32),
                   jax.ShapeDtypeStruct((P, m1, C), _F32),
                   jax.ShapeDtypeStruct((P, m, C), _F32),
                   jax.ShapeDtypeStruct((P, 1, C), _F32)],
    )(xw, lng.reshape(1, C), lnb.reshape(1, C), wq, wk, wv,
      bq.reshape(1, C), bk.reshape(1, C), bv.reshape(1, C), pmat)
    return out


def _route(qm, km, topk):
    P, C = qm.shape

    def body(qm_ref, km_ref, idx_ref):
        q = qm_ref[...]
        k = km_ref[...]
        a = lax.dot_general(q, k, (((1,), (1,)), ((), ())),
                            preferred_element_type=_F32)
        col = lax.broadcasted_iota(jnp.int32, (P, P), 1)
        cols = []
        for _ in range(topk):
            mx = jnp.max(a, axis=1, keepdims=True)
            am = jnp.min(jnp.where(a >= mx, col, P), axis=1, keepdims=True)
            cols.append(am)
            a = jnp.where(col == am, -jnp.inf, a)
        idx_ref[...] = jnp.concatenate(cols, axis=1)

    return _pcall(
        body,
        out_shape=jax.ShapeDtypeStruct((P, topk), jnp.int32),
    )(qm, km)


def _attn(idxf, qw, kp, vp, nh, m, topk):
    P, hw, C = qw.shape
    c = C // nh
    scale = c ** -0.5

    def body(idx_ref, qw_ref, kp_ref, vp_ref, ow_ref):
        i = pl.program_id(0)
        q = qw_ref[0]
        ks_parts, vs_parts = [], []
        for t in range(topk):
            j = idx_ref[i * topk + t]
            ks_parts.append(kp_ref[pl.ds(j * m, m), :])
            vs_parts.append(vp_ref[pl.ds(j * m, m), :])
        ks = jnp.concatenate(ks_parts, 0) if topk > 1 else ks_parts[0]
        vs = jnp.concatenate(vs_parts, 0) if topk > 1 else vs_parts[0]
        outs = []
        for hh in range(nh):
            qh = q[:, hh * c:(hh + 1) * c] * scale
            kh = ks[:, hh * c:(hh + 1) * c]
            vh = vs[:, hh * c:(hh + 1) * c]
            logits = lax.dot_general(qh, kh, (((1,), (1,)), ((), ())),
                                     preferred_element_type=_F32)
            pa = jax.nn.softmax(logits, axis=-1)
            outs.append(jnp.dot(pa, vh, preferred_element_type=_F32))
        ow_ref[0] = jnp.concatenate(outs, 1) if nh > 1 else outs[0]

    grid_spec = pltpu.PrefetchScalarGridSpec(
        num_scalar_prefetch=1,
        grid=(P,),
        in_specs=[pl.BlockSpec((1, hw, C), lambda i, idx: (i, 0, 0)),
                  pl.BlockSpec((P * m, C), lambda i, idx: (0, 0)),
                  pl.BlockSpec((P * m, C), lambda i, idx: (0, 0))],
        out_specs=pl.BlockSpec((1, hw, C), lambda i, idx: (i, 0, 0)),
    )
    return _pcall(
        body,
        grid_spec=grid_spec,
        out_shape=jax.ShapeDtypeStruct((P, hw, C), _F32),
    )(idxf, qw, kp, vp)


def _wo_res(xres, aw, lepe, wo, wob):
    HW, C = xres.shape
    R = 256 if HW % 256 == 0 else (112 if HW % 112 == 0 else HW)

    def body(x_ref, a_ref, l_ref, w_ref, b_ref, o_ref):
        y = a_ref[...] + l_ref[...]
        o_ref[...] = x_ref[...] + jnp.dot(
            y, w_ref[...], preferred_element_type=_F32) + b_ref[...]

    return _pcall(
        body,
        grid=(HW // R,),
        in_specs=[pl.BlockSpec((R, C), lambda i: (i, 0)),
                  pl.BlockSpec((R, C), lambda i: (i, 0)),
                  pl.BlockSpec((R, C), lambda i: (i, 0)),
                  pl.BlockSpec((C, C), lambda i: (0, 0)),
                  pl.BlockSpec((1, C), lambda i: (0, 0))],
        out_specs=pl.BlockSpec((R, C), lambda i: (i, 0)),
        out_shape=jax.ShapeDtypeStruct((HW, C), _F32),
    )(xres, aw, lepe, wo, wob.reshape(1, C))


def _mlp(x, g, b, w1, b1, w2, b2):
    HW, C = x.shape
    C2 = w1.shape[1]
    R = 256 if HW % 256 == 0 else (112 if HW % 112 == 0 else HW)

    def body(x_ref, g_ref, b_ref, w1_ref, b1_ref, w2_ref, b2_ref, o_ref):
        x_ = x_ref[...]
        mu = jnp.mean(x_, axis=-1, keepdims=True)
        xc = x_ - mu
        var = jnp.mean(xc * xc, axis=-1, keepdims=True)
        xn = xc * lax.rsqrt(var + 1e-5) * g_ref[...] + b_ref[...]
        h = jax.nn.gelu(jnp.dot(xn, w1_ref[...], preferred_element_type=_F32)
                        + b1_ref[...])
        o_ref[...] = x_ + jnp.dot(h, w2_ref[...],
                                  preferred_element_type=_F32) + b2_ref[...]

    return _pcall(
        body,
        grid=(HW // R,),
        in_specs=[pl.BlockSpec((R, C), lambda i: (i, 0)),
                  pl.BlockSpec((1, C), lambda i: (0, 0)),
                  pl.BlockSpec((1, C), lambda i: (0, 0)),
                  pl.BlockSpec((C, C2), lambda i: (0, 0)),
                  pl.BlockSpec((1, C2), lambda i: (0, 0)),
                  pl.BlockSpec((C2, C), lambda i: (0, 0)),
                  pl.BlockSpec((1, C), lambda i: (0, 0))],
        out_specs=pl.BlockSpec((R, C), lambda i: (i, 0)),
        out_shape=jax.ShapeDtypeStruct((HW, C), _F32),
    )(x, g.reshape(1, C), b.reshape(1, C), w1, b1.reshape(1, C2),
      w2, b2.reshape(1, C))


def _bra(x, p, n_win, nh, topk, kv_per_win):
    _, H, W, C = x.shape
    h, w = H // n_win, W // n_win
    P = n_win * n_win
    m = kv_per_win * kv_per_win
    hw = h * w

    x2 = x[0]
    xw = _win(x2, n_win, h, w, C)
    wq, wk, wv = jnp.split(p['qkv_w'], 3, axis=1)
    bq, bk, bv = jnp.split(p['qkv_b'], 3)
    pmat = _pool_matrix(h, w, kv_per_win)

    qw, vw, kpm, vp, qm = _stage_a(
        xw, p['ln1_g'], p['ln1_b'], wq, wk, wv, bq, bk, bv, pmat)

    km = kpm[:, m, :]
    kp = kpm[:, :m, :].reshape(P * m, C)
    vpf = vp.reshape(P * m, C)

    idx = _route(qm[:, 0, :], km, topk)
    idxf = idx.reshape(P * topk)

    aw = _attn(idxf, qw, kp, vpf, nh, m, topk)

    v_spat = _unwin(vw, n_win, h, w, C)
    lepe = _conv2d(v_spat.transpose(2, 0, 1)[None], p['lepe_w'],
                   groups=C)[0].transpose(1, 2, 0)
    a_spat = _unwin(aw, n_win, h, w, C)

    o = _wo_res(x2.reshape(H * W, C), a_spat.reshape(H * W, C),
                lepe.reshape(H * W, C), p['wo_w'], p['wo_b'])
    return o.reshape(1, H, W, C)


def _block(x, p, n_win, nh, topk, kv_per_win):
    C = x.shape[1]
    x = x + _conv2d(x, p['cpe_w'], groups=C) + p['cpe_b'][None, :, None, None]
    x = x.transpose(0, 2, 3, 1)
    _, H, W, _ = x.shape
    x = _bra(x, p, n_win, nh, topk, kv_per_win)
    xf = _mlp(x.reshape(H * W, C), p['ln2_g'], p['ln2_b'],
              p['mlp_w1'], p['mlp_b1'], p['mlp_w2'], p['mlp_b2'])
    return xf.reshape(1, H, W, C).transpose(0, 3, 1, 2)


def kernel(img0, img1, warped_img0, warped_img1, mask, flow, c0_0, c0_1,
           c0_2, c0_3, c1_0, c1_1, c1_2, c1_3, mask_guide_0, mask_guide_1,
           mask_guide_2, params):
    p = params
    inp = jnp.concatenate([img0, img1, mask, mask_guide_0, warped_img0,
                           warped_img1, c0_0, c1_0, flow], 1)
    x1 = _conv2d(inp, p['patch_embed'])
    for bp in p['enc1']:
        x1 = _block(x1, bp, 14, 1, 6, 2)
    f1 = _conv2d(jnp.concatenate([mask_guide_1, c0_1, c1_1], 1), p['c_down1'])
    x2 = _pixel_unshuffle(_conv2d(x1, p['down1_2']), 2)
    x2 = jnp.concatenate([x2, f1], 1)
    for bp in p['enc2']:
        x2 = _block(x2, bp, 7, 2, 4, 1)
    f2 = _conv2d(jnp.concatenate([mask_guide_2, c0_2, c1_2], 1), p['c_down2'])
    x3 = _pixel_unshuffle(_conv2d(x2, p['down2_3']), 2)
    x3 = jnp.concatenate([x3, f2], 1)
    for bp in p['enc3']:
        x3 = _block(x3, bp, 7, 4, 4, 1)
    f3 = _conv2d(jnp.concatenate([c0_3, c1_3], 1), p['c_down3'])
    x4 = _pixel_unshuffle(_conv2d(x3, p['down3_4']), 2)
    x4 = jnp.concatenate([x4, f3], 1)
    for bp in p['latent']:
        x4 = _block(x4, bp, 7, 8, 4, 1)
    d3 = _pixel_shuffle(_conv2d(x4, p['up4_3']), 2)
    d3 = _conv2d(jnp.concatenate([d3, x3], 1), p['reduce3'])
    for bp in p['dec3']:
        d3 = _block(d3, bp, 7, 1, 4, 1)
    d2 = _pixel_shuffle(_conv2d(d3, p['up3_2']), 2)
    d2 = _conv2d(jnp.concatenate([d2, x2], 1), p['reduce2'])
    for bp in p['dec2']:
        d2 = _block(d2, bp, 7, 1, 4, 1)
    d1 = _pixel_shuffle(_conv2d(d2, p['up2_1']), 2)
    d1 = jnp.concatenate([d1, x1], 1)
    for bp in p['dec1']:
        d1 = _block(d1, bp, 7, 1, 4, 2)
    for bp in p['refine']:
        d1 = _block(d1, bp, 14, 1, 6, 2)
    return jax.nn.sigmoid(_conv2d(d1, p['out_w']))


# speedup vs baseline: 1.3930x; 1.3930x over previous
import functools
import numpy as np
import jax
import jax.numpy as jnp
from jax import lax
from jax.experimental import pallas as pl
from jax.experimental.pallas import tpu as pltpu

_F32 = jnp.float32


def _pcall(body, **kw):
    return pl.pallas_call(body, **kw)


def _rows(HW, C):
    for r in (2048, 1024, 784, 512, 448, 392, 256, 224, 112, 98, 56, 49, 28, 16, 8):
        if HW % r == 0 and r * C * 4 <= 1 << 20:
            return r
    return HW


def _conv(x, w, groups=1):
    return lax.conv_general_dilated(
        x, w.transpose(2, 3, 1, 0), (1, 1), 'SAME',
        dimension_numbers=('NHWC', 'HWIO', 'NHWC'),
        feature_group_count=groups)


def _pixel_unshuffle(x, r):
    B, H, W, C = x.shape
    x = x.reshape(B, H // r, r, W // r, r, C)
    return x.transpose(0, 1, 3, 5, 2, 4).reshape(B, H // r, W // r, C * r * r)


def _pixel_shuffle(x, r):
    B, H, W, C = x.shape
    x = x.reshape(B, H, W, C // (r * r), r, r)
    return x.transpose(0, 1, 4, 2, 5, 3).reshape(B, H * r, W * r, C // (r * r))


def _win(t, n_win, h, w, C):
    return (t.reshape(n_win, h, n_win, w, C)
             .transpose(0, 2, 1, 3, 4)
             .reshape(n_win * n_win, h * w, C))


def _unwin(t, n_win, h, w, C):
    return (t.reshape(n_win, n_win, h, w, C)
             .transpose(0, 2, 1, 3, 4)
             .reshape(n_win * h, n_win * w, C))


def _ln_qv(x, g, b, wq, wv, bq, bv):
    HW, C = x.shape
    R = _rows(HW, C)

    def body(x_ref, g_ref, b_ref, wq_ref, wv_ref, bq_ref, bv_ref,
             xn_ref, q_ref, v_ref):
        x_ = x_ref[...]
        mu = jnp.mean(x_, axis=-1, keepdims=True)
        xc = x_ - mu
        var = jnp.mean(xc * xc, axis=-1, keepdims=True)
        xn = xc * lax.rsqrt(var + 1e-5) * g_ref[...] + b_ref[...]
        xn_ref[...] = xn
        q_ref[...] = jnp.dot(xn, wq_ref[...], preferred_element_type=_F32) + bq_ref[...]
        v_ref[...] = jnp.dot(xn, wv_ref[...], preferred_element_type=_F32) + bv_ref[...]

    full = lambda shape: pl.BlockSpec(shape, lambda i: (0,) * len(shape))
    row = pl.BlockSpec((R, C), lambda i: (i, 0))
    return _pcall(
        body,
        grid=(HW // R,),
        in_specs=[row, full((1, C)), full((1, C)), full((C, C)), full((C, C)),
                  full((1, C)), full((1, C))],
        out_specs=[row, row, row],
        out_shape=[jax.ShapeDtypeStruct((HW, C), _F32)] * 3,
    )(x, g.reshape(1, C), b.reshape(1, C), wq, wv,
      bq.reshape(1, C), bv.reshape(1, C))


def _route_proj(xp, xm, wq, wk, wv, bq, bk, bv, topk):
    Pm, C = xp.shape
    P = xm.shape[0]

    def body(xp_ref, xm_ref, wq_ref, wk_ref, wv_ref, bq_ref, bk_ref, bv_ref,
             kp_ref, vp_ref, idx_ref):
        xp_ = xp_ref[...]
        xm_ = xm_ref[...]
        kp_ref[...] = jnp.dot(xp_, wk_ref[...], preferred_element_type=_F32) + bk_ref[...]
        vp_ref[...] = jnp.dot(xp_, wv_ref[...], preferred_element_type=_F32) + bv_ref[...]
        qm = jnp.dot(xm_, wq_ref[...], preferred_element_type=_F32) + bq_ref[...]
        km = jnp.dot(xm_, wk_ref[...], preferred_element_type=_F32) + bk_ref[...]
        a = lax.dot_general(qm, km, (((1,), (1,)), ((), ())),
                            preferred_element_type=_F32)
        col = lax.broadcasted_iota(jnp.int32, (P, P), 1)
        cols = []
        for _ in range(topk):
            mx = jnp.max(a, axis=1, keepdims=True)
            am = jnp.min(jnp.where(a >= mx, col, P), axis=1, keepdims=True)
            cols.append(am)
            a = jnp.where(col == am, -jnp.inf, a)
        idx_ref[...] = jnp.concatenate(cols, axis=1)

    return _pcall(
        body,
        out_shape=[jax.ShapeDtypeStruct((Pm, C), _F32),
                   jax.ShapeDtypeStruct((Pm, C), _F32),
                   jax.ShapeDtypeStruct((P, topk), jnp.int32)],
    )(xp, xm, wq, wk, wv, bq.reshape(1, C), bk.reshape(1, C), bv.reshape(1, C))


def _attn(idxf, qw, kp, vp, nh, m, topk):
    P, hw, C = qw.shape
    c = C // nh
    scale = c ** -0.5

    def body(idx_ref, qw_ref, kp_ref, vp_ref, ow_ref):
        i = pl.program_id(0)
        q = qw_ref[0]
        ks_parts, vs_parts = [], []
        for t in range(topk):
            j = idx_ref[i * topk + t]
            ks_parts.append(kp_ref[pl.ds(j * m, m), :])
            vs_parts.append(vp_ref[pl.ds(j * m, m), :])
        ks = jnp.concatenate(ks_parts, 0) if topk > 1 else ks_parts[0]
        vs = jnp.concatenate(vs_parts, 0) if topk > 1 else vs_parts[0]
        outs = []
        for hh in range(nh):
            qh = q[:, hh * c:(hh + 1) * c] * scale
            kh = ks[:, hh * c:(hh + 1) * c]
            vh = vs[:, hh * c:(hh + 1) * c]
            logits = lax.dot_general(qh, kh, (((1,), (1,)), ((), ())),
                                     preferred_element_type=_F32)
            pa = jax.nn.softmax(logits, axis=-1)
            outs.append(jnp.dot(pa, vh, preferred_element_type=_F32))
        ow_ref[0] = jnp.concatenate(outs, 1) if nh > 1 else outs[0]

    grid_spec = pltpu.PrefetchScalarGridSpec(
        num_scalar_prefetch=1,
        grid=(P,),
        in_specs=[pl.BlockSpec((1, hw, C), lambda i, idx: (i, 0, 0)),
                  pl.BlockSpec((P * m, C), lambda i, idx: (0, 0)),
                  pl.BlockSpec((P * m, C), lambda i, idx: (0, 0))],
        out_specs=pl.BlockSpec((1, hw, C), lambda i, idx: (i, 0, 0)),
    )
    return _pcall(
        body,
        grid_spec=grid_spec,
        out_shape=jax.ShapeDtypeStruct((P, hw, C), _F32),
    )(idxf, qw, kp, vp)


def _wo_mlp(xres, aw, lepe, wo, wob, g, b, w1, b1, w2, b2):
    HW, C = xres.shape
    C2 = w1.shape[1]
    R = _rows(HW, C)

    def body(x_ref, a_ref, l_ref, wo_ref, wob_ref, g_ref, b_ref,
             w1_ref, b1_ref, w2_ref, b2_ref, o_ref):
        y = a_ref[...] + l_ref[...]
        x1 = x_ref[...] + jnp.dot(y, wo_ref[...],
                                  preferred_element_type=_F32) + wob_ref[...]
        mu = jnp.mean(x1, axis=-1, keepdims=True)
        xc = x1 - mu
        var = jnp.mean(xc * xc, axis=-1, keepdims=True)
        xn = xc * lax.rsqrt(var + 1e-5) * g_ref[...] + b_ref[...]
        h = jax.nn.gelu(jnp.dot(xn, w1_ref[...], preferred_element_type=_F32)
                        + b1_ref[...])
        o_ref[...] = x1 + jnp.dot(h, w2_ref[...],
                                  preferred_element_type=_F32) + b2_ref[...]

    full = lambda shape: pl.BlockSpec(shape, lambda i: (0,) * len(shape))
    row = pl.BlockSpec((R, C), lambda i: (i, 0))
    return _pcall(
        body,
        grid=(HW // R,),
        in_specs=[row, row, row, full((C, C)), full((1, C)), full((1, C)),
                  full((1, C)), full((C, C2)), full((1, C2)), full((C2, C)),
                  full((1, C))],
        out_specs=row,
        out_shape=jax.ShapeDtypeStruct((HW, C), _F32),
    )(xres, aw, lepe, wo, wob.reshape(1, C), g.reshape(1, C), b.reshape(1, C),
      w1, b1.reshape(1, C2), w2, b2.reshape(1, C))


def _block(x, p, n_win, nh, topk, kv_per_win):
    C = x.shape[-1]
    x = x + _conv(x, p['cpe_w'], groups=C) + p['cpe_b']
    _, H, W, _ = x.shape
    h, w = H // n_win, W // n_win
    P = n_win * n_win
    m = kv_per_win * kv_per_win
    bh, bw = h // kv_per_win, w // kv_per_win
    HW = H * W

    wq, wk, wv = jnp.split(p['qkv_w'], 3, axis=1)
    bq, bk, bv = jnp.split(p['qkv_b'], 3)

    x2 = x[0]
    xn, q, v = _ln_qv(x2.reshape(HW, C), p['ln1_g'], p['ln1_b'],
                      wq, wv, bq, bv)

    xn3 = xn.reshape(H, W, C)
    xp = (xn3.reshape(n_win, kv_per_win, bh, n_win, kv_per_win, bw, C)
             .mean(axis=(2, 5))
             .transpose(0, 2, 1, 3, 4)
             .reshape(P * m, C))
    xm = xn3.reshape(n_win, h, n_win, w, C).mean(axis=(1, 3)).reshape(P, C)

    kp, vp, idx = _route_proj(xp, xm, wq, wk, wv, bq, bk, bv, topk)

    qw = _win(q.reshape(H, W, C), n_win, h, w, C)
    aw = _attn(idx.reshape(P * topk), qw, kp, vp, nh, m, topk)
    a_spat = _unwin(aw, n_win, h, w, C).reshape(HW, C)

    lepe = _conv(v.reshape(1, H, W, C), p['lepe_w'], groups=C)[0].reshape(HW, C)

    out = _wo_mlp(x2.reshape(HW, C), a_spat, lepe, p['wo_w'], p['wo_b'],
                  p['ln2_g'], p['ln2_b'], p['mlp_w1'], p['mlp_b1'],
                  p['mlp_w2'], p['mlp_b2'])
    return out.reshape(1, H, W, C)


def kernel(img0, img1, warped_img0, warped_img1, mask, flow, c0_0, c0_1,
           c0_2, c0_3, c1_0, c1_1, c1_2, c1_3, mask_guide_0, mask_guide_1,
           mask_guide_2, params):
    p = params
    nhwc = lambda t: t.transpose(0, 2, 3, 1)
    inp = jnp.concatenate([img0, img1, mask, mask_guide_0, warped_img0,
                           warped_img1, c0_0, c1_0, flow], 1)
    x1 = _conv(nhwc(inp), p['patch_embed'])
    for bp in p['enc1']:
        x1 = _block(x1, bp, 14, 1, 6, 2)
    f1 = _conv(nhwc(jnp.concatenate([mask_guide_1, c0_1, c1_1], 1)),
               p['c_down1'])
    x2 = _pixel_unshuffle(_conv(x1, p['down1_2']), 2)
    x2 = jnp.concatenate([x2, f1], -1)
    for bp in p['enc2']:
        x2 = _block(x2, bp, 7, 2, 4, 1)
    f2 = _conv(nhwc(jnp.concatenate([mask_guide_2, c0_2, c1_2], 1)),
               p['c_down2'])
    x3 = _pixel_unshuffle(_conv(x2, p['down2_3']), 2)
    x3 = jnp.concatenate([x3, f2], -1)
    for bp in p['enc3']:
        x3 = _block(x3, bp, 7, 4, 4, 1)
    f3 = _conv(nhwc(jnp.concatenate([c0_3, c1_3], 1)), p['c_down3'])
    x4 = _pixel_unshuffle(_conv(x3, p['down3_4']), 2)
    x4 = jnp.concatenate([x4, f3], -1)
    for bp in p['latent']:
        x4 = _block(x4, bp, 7, 8, 4, 1)
    d3 = _pixel_shuffle(_conv(x4, p['up4_3']), 2)
    d3 = _conv(jnp.concatenate([d3, x3], -1), p['reduce3'])
    for bp in p['dec3']:
        d3 = _block(d3, bp, 7, 1, 4, 1)
    d2 = _pixel_shuffle(_conv(d3, p['up3_2']), 2)
    d2 = _conv(jnp.concatenate([d2, x2], -1), p['reduce2'])
    for bp in p['dec2']:
        d2 = _block(d2, bp, 7, 1, 4, 1)
    d1 = _pixel_shuffle(_conv(d2, p['up2_1']), 2)
    d1 = jnp.concatenate([d1, x1], -1)
    for bp in p['dec1']:
        d1 = _block(d1, bp, 7, 1, 4, 2)
    for bp in p['refine']:
        d1 = _block(d1, bp, 14, 1, 6, 2)
    return jax.nn.sigmoid(_conv(d1, p['out_w'])).transpose(0, 3, 1, 2)


# speedup vs baseline: 1.6247x; 1.1664x over previous
import functools
import numpy as np
import jax
import jax.numpy as jnp
from jax import lax
from jax.experimental import pallas as pl
from jax.experimental.pallas import tpu as pltpu

_F32 = jnp.float32


def _pcall(body, **kw):
    return pl.pallas_call(body, **kw)


def _rows(HW, C):
    for r in (2048, 1024, 784, 512, 448, 392, 256, 224, 112, 98, 56, 49, 28, 16, 8):
        if HW % r == 0 and r * C * 4 <= 1 << 20:
            return r
    return HW


def _conv(x, w, groups=1):
    return lax.conv_general_dilated(
        x, w.transpose(2, 3, 1, 0), (1, 1), 'SAME',
        dimension_numbers=('NHWC', 'HWIO', 'NHWC'),
        feature_group_count=groups)


def _pixel_unshuffle(x, r):
    B, H, W, C = x.shape
    x = x.reshape(B, H // r, r, W // r, r, C)
    return x.transpose(0, 1, 3, 5, 2, 4).reshape(B, H // r, W // r, C * r * r)


def _pixel_shuffle(x, r):
    B, H, W, C = x.shape
    x = x.reshape(B, H, W, C // (r * r), r, r)
    return x.transpose(0, 1, 4, 2, 5, 3).reshape(B, H * r, W * r, C // (r * r))


def _win(t, n_win, h, w, C):
    return (t.reshape(n_win, h, n_win, w, C)
             .transpose(0, 2, 1, 3, 4)
             .reshape(n_win * n_win, h * w, C))


def _unwin(t, n_win, h, w, C):
    return (t.reshape(n_win, n_win, h, w, C)
             .transpose(0, 2, 1, 3, 4)
             .reshape(n_win * h, n_win * w, C))


def _ln_qv(x, g, b, wq, wv, bq, bv):
    HW, C = x.shape
    R = _rows(HW, C)

    def body(x_ref, g_ref, b_ref, wq_ref, wv_ref, bq_ref, bv_ref,
             xn_ref, q_ref, v_ref):
        x_ = x_ref[...]
        mu = jnp.mean(x_, axis=-1, keepdims=True)
        xc = x_ - mu
        var = jnp.mean(xc * xc, axis=-1, keepdims=True)
        xn = xc * lax.rsqrt(var + 1e-5) * g_ref[...] + b_ref[...]
        xn_ref[...] = xn
        q_ref[...] = jnp.dot(xn, wq_ref[...], preferred_element_type=_F32) + bq_ref[...]
        v_ref[...] = jnp.dot(xn, wv_ref[...], preferred_element_type=_F32) + bv_ref[...]

    full = lambda shape: pl.BlockSpec(shape, lambda i: (0,) * len(shape))
    row = pl.BlockSpec((R, C), lambda i: (i, 0))
    return _pcall(
        body,
        grid=(HW // R,),
        in_specs=[row, full((1, C)), full((1, C)), full((C, C)), full((C, C)),
                  full((1, C)), full((1, C))],
        out_specs=[row, row, row],
        out_shape=[jax.ShapeDtypeStruct((HW, C), _F32)] * 3,
    )(x, g.reshape(1, C), b.reshape(1, C), wq, wv,
      bq.reshape(1, C), bv.reshape(1, C))


def _route_proj(xp, xm, wq, wk, wv, bq, bk, bv, topk):
    Pm, C = xp.shape
    P = xm.shape[0]

    def body(xp_ref, xm_ref, wq_ref, wk_ref, wv_ref, bq_ref, bk_ref, bv_ref,
             kp_ref, vp_ref, idx_ref):
        xp_ = xp_ref[...]
        xm_ = xm_ref[...]
        kp_ref[...] = jnp.dot(xp_, wk_ref[...], preferred_element_type=_F32) + bk_ref[...]
        vp_ref[...] = jnp.dot(xp_, wv_ref[...], preferred_element_type=_F32) + bv_ref[...]
        qm = jnp.dot(xm_, wq_ref[...], preferred_element_type=_F32) + bq_ref[...]
        km = jnp.dot(xm_, wk_ref[...], preferred_element_type=_F32) + bk_ref[...]
        a = lax.dot_general(qm, km, (((1,), (1,)), ((), ())),
                            preferred_element_type=_F32)
        col = lax.broadcasted_iota(jnp.int32, (P, P), 1)
        cols = []
        for _ in range(topk):
            mx = jnp.max(a, axis=1, keepdims=True)
            am = jnp.min(jnp.where(a >= mx, col, P), axis=1, keepdims=True)
            cols.append(am)
            a = jnp.where(col == am, -jnp.inf, a)
        idx_ref[...] = jnp.concatenate(cols, axis=1)

    return _pcall(
        body,
        out_shape=[jax.ShapeDtypeStruct((Pm, C), _F32),
                   jax.ShapeDtypeStruct((Pm, C), _F32),
                   jax.ShapeDtypeStruct((P, topk), jnp.int32)],
    )(xp, xm, wq, wk, wv, bq.reshape(1, C), bk.reshape(1, C), bv.reshape(1, C))


def _attn(idxf, qw, kp, vp, nh, m, topk):
    P, hw, C = qw.shape
    c = C // nh
    scale = c ** -0.5
    G = 14 if P % 14 == 0 else (7 if P % 7 == 0 else 1)

    def body(idx_ref, qw_ref, kp_ref, vp_ref, ow_ref):
        i = pl.program_id(0)
        for g in range(G):
            q = qw_ref[g]
            ks_parts, vs_parts = [], []
            for t in range(topk):
                j = idx_ref[(i * G + g) * topk + t]
                ks_parts.append(kp_ref[pl.ds(j * m, m), :])
                vs_parts.append(vp_ref[pl.ds(j * m, m), :])
            ks = jnp.concatenate(ks_parts, 0) if topk > 1 else ks_parts[0]
            vs = jnp.concatenate(vs_parts, 0) if topk > 1 else vs_parts[0]
            outs = []
            for hh in range(nh):
                qh = q[:, hh * c:(hh + 1) * c] * scale
                kh = ks[:, hh * c:(hh + 1) * c]
                vh = vs[:, hh * c:(hh + 1) * c]
                logits = lax.dot_general(qh, kh, (((1,), (1,)), ((), ())),
                                         preferred_element_type=_F32)
                pa = jax.nn.softmax(logits, axis=-1)
                outs.append(jnp.dot(pa, vh, preferred_element_type=_F32))
            ow_ref[g] = jnp.concatenate(outs, 1) if nh > 1 else outs[0]

    grid_spec = pltpu.PrefetchScalarGridSpec(
        num_scalar_prefetch=1,
        grid=(P // G,),
        in_specs=[pl.BlockSpec((G, hw, C), lambda i, idx: (i, 0, 0)),
                  pl.BlockSpec((P * m, C), lambda i, idx: (0, 0)),
                  pl.BlockSpec((P * m, C), lambda i, idx: (0, 0))],
        out_specs=pl.BlockSpec((G, hw, C), lambda i, idx: (i, 0, 0)),
    )
    return _pcall(
        body,
        grid_spec=grid_spec,
        out_shape=jax.ShapeDtypeStruct((P, hw, C), _F32),
    )(idxf, qw, kp, vp)


def _wo_mlp(xres, aw, lepe, wo, wob, g, b, w1, b1, w2, b2):
    HW, C = xres.shape
    C2 = w1.shape[1]
    R = _rows(HW, C)

    def body(x_ref, a_ref, l_ref, wo_ref, wob_ref, g_ref, b_ref,
             w1_ref, b1_ref, w2_ref, b2_ref, o_ref):
        y = a_ref[...] + l_ref[...]
        x1 = x_ref[...] + jnp.dot(y, wo_ref[...],
                                  preferred_element_type=_F32) + wob_ref[...]
        mu = jnp.mean(x1, axis=-1, keepdims=True)
        xc = x1 - mu
        var = jnp.mean(xc * xc, axis=-1, keepdims=True)
        xn = xc * lax.rsqrt(var + 1e-5) * g_ref[...] + b_ref[...]
        h = jax.nn.gelu(jnp.dot(xn, w1_ref[...], preferred_element_type=_F32)
                        + b1_ref[...])
        o_ref[...] = x1 + jnp.dot(h, w2_ref[...],
                                  preferred_element_type=_F32) + b2_ref[...]

    full = lambda shape: pl.BlockSpec(shape, lambda i: (0,) * len(shape))
    row = pl.BlockSpec((R, C), lambda i: (i, 0))
    return _pcall(
        body,
        grid=(HW // R,),
        in_specs=[row, row, row, full((C, C)), full((1, C)), full((1, C)),
                  full((1, C)), full((C, C2)), full((1, C2)), full((C2, C)),
                  full((1, C))],
        out_specs=row,
        out_shape=jax.ShapeDtypeStruct((HW, C), _F32),
    )(xres, aw, lepe, wo, wob.reshape(1, C), g.reshape(1, C), b.reshape(1, C),
      w1, b1.reshape(1, C2), w2, b2.reshape(1, C))


def _block(x, p, n_win, nh, topk, kv_per_win):
    C = x.shape[-1]
    x = x + _conv(x, p['cpe_w'], groups=C) + p['cpe_b']
    _, H, W, _ = x.shape
    h, w = H // n_win, W // n_win
    P = n_win * n_win
    m = kv_per_win * kv_per_win
    bh, bw = h // kv_per_win, w // kv_per_win
    HW = H * W

    wq, wk, wv = jnp.split(p['qkv_w'], 3, axis=1)
    bq, bk, bv = jnp.split(p['qkv_b'], 3)

    x2 = x[0]
    xn, q, v = _ln_qv(x2.reshape(HW, C), p['ln1_g'], p['ln1_b'],
                      wq, wv, bq, bv)

    xn3 = xn.reshape(H, W, C)
    xp = (xn3.reshape(n_win, kv_per_win, bh, n_win, kv_per_win, bw, C)
             .mean(axis=(2, 5))
             .transpose(0, 2, 1, 3, 4)
             .reshape(P * m, C))
    xm = xn3.reshape(n_win, h, n_win, w, C).mean(axis=(1, 3)).reshape(P, C)

    kp, vp, idx = _route_proj(xp, xm, wq, wk, wv, bq, bk, bv, topk)

    qw = _win(q.reshape(H, W, C), n_win, h, w, C)
    aw = _attn(idx.reshape(P * topk), qw, kp, vp, nh, m, topk)
    a_spat = _unwin(aw, n_win, h, w, C).reshape(HW, C)

    lepe = _conv(v.reshape(1, H, W, C), p['lepe_w'], groups=C)[0].reshape(HW, C)

    out = _wo_mlp(x2.reshape(HW, C), a_spat, lepe, p['wo_w'], p['wo_b'],
                  p['ln2_g'], p['ln2_b'], p['mlp_w1'], p['mlp_b1'],
                  p['mlp_w2'], p['mlp_b2'])
    return out.reshape(1, H, W, C)


def kernel(img0, img1, warped_img0, warped_img1, mask, flow, c0_0, c0_1,
           c0_2, c0_3, c1_0, c1_1, c1_2, c1_3, mask_guide_0, mask_guide_1,
           mask_guide_2, params):
    p = params
    nhwc = lambda t: t.transpose(0, 2, 3, 1)
    inp = jnp.concatenate([img0, img1, mask, mask_guide_0, warped_img0,
                           warped_img1, c0_0, c1_0, flow], 1)
    x1 = _conv(nhwc(inp), p['patch_embed'])
    for bp in p['enc1']:
        x1 = _block(x1, bp, 14, 1, 6, 2)
    f1 = _conv(nhwc(jnp.concatenate([mask_guide_1, c0_1, c1_1], 1)),
               p['c_down1'])
    x2 = _pixel_unshuffle(_conv(x1, p['down1_2']), 2)
    x2 = jnp.concatenate([x2, f1], -1)
    for bp in p['enc2']:
        x2 = _block(x2, bp, 7, 2, 4, 1)
    f2 = _conv(nhwc(jnp.concatenate([mask_guide_2, c0_2, c1_2], 1)),
               p['c_down2'])
    x3 = _pixel_unshuffle(_conv(x2, p['down2_3']), 2)
    x3 = jnp.concatenate([x3, f2], -1)
    for bp in p['enc3']:
        x3 = _block(x3, bp, 7, 4, 4, 1)
    f3 = _conv(nhwc(jnp.concatenate([c0_3, c1_3], 1)), p['c_down3'])
    x4 = _pixel_unshuffle(_conv(x3, p['down3_4']), 2)
    x4 = jnp.concatenate([x4, f3], -1)
    for bp in p['latent']:
        x4 = _block(x4, bp, 7, 8, 4, 1)
    d3 = _pixel_shuffle(_conv(x4, p['up4_3']), 2)
    d3 = _conv(jnp.concatenate([d3, x3], -1), p['reduce3'])
    for bp in p['dec3']:
        d3 = _block(d3, bp, 7, 1, 4, 1)
    d2 = _pixel_shuffle(_conv(d3, p['up3_2']), 2)
    d2 = _conv(jnp.concatenate([d2, x2], -1), p['reduce2'])
    for bp in p['dec2']:
        d2 = _block(d2, bp, 7, 1, 4, 1)
    d1 = _pixel_shuffle(_conv(d2, p['up2_1']), 2)
    d1 = jnp.concatenate([d1, x1], -1)
    for bp in p['dec1']:
        d1 = _block(d1, bp, 7, 1, 4, 2)
    for bp in p['refine']:
        d1 = _block(d1, bp, 14, 1, 6, 2)
    return jax.nn.sigmoid(_conv(d1, p['out_w'])).transpose(0, 3, 1, 2)


# speedup vs baseline: 1.6950x; 1.0433x over previous
import functools
import numpy as np
import jax
import jax.numpy as jnp
from jax import lax
from jax.experimental import pallas as pl
from jax.experimental.pallas import tpu as pltpu

_F32 = jnp.float32


def _pcall(body, **kw):
    return pl.pallas_call(body, **kw)


def _rows(HW, C):
    for r in (2048, 1024, 784, 512, 448, 392, 256, 224, 112, 98, 56, 49, 28, 16, 8):
        if HW % r == 0 and r * C * 4 <= 1 << 20:
            return r
    return HW


def _conv(x, w, groups=1):
    return lax.conv_general_dilated(
        x, w.transpose(2, 3, 1, 0), (1, 1), 'SAME',
        dimension_numbers=('NHWC', 'HWIO', 'NHWC'),
        feature_group_count=groups)


def _pixel_unshuffle(x, r):
    B, H, W, C = x.shape
    x = x.reshape(B, H // r, r, W // r, r, C)
    return x.transpose(0, 1, 3, 5, 2, 4).reshape(B, H // r, W // r, C * r * r)


def _pixel_shuffle(x, r):
    B, H, W, C = x.shape
    x = x.reshape(B, H, W, C // (r * r), r, r)
    return x.transpose(0, 1, 4, 2, 5, 3).reshape(B, H * r, W * r, C // (r * r))


def _win(t, n_win, h, w, C):
    return (t.reshape(n_win, h, n_win, w, C)
             .transpose(0, 2, 1, 3, 4)
             .reshape(n_win * n_win, h * w, C))


def _unwin(t, n_win, h, w, C):
    return (t.reshape(n_win, n_win, h, w, C)
             .transpose(0, 2, 1, 3, 4)
             .reshape(n_win * h, n_win * w, C))


def _dwconv(x, w, k, bias=None, residual=False):
    H, W, C = x.shape
    p = k // 2
    Wp = W + 2 * p
    Rh = 28
    wf = w.transpose(2, 3, 1, 0).reshape(k * k, C)
    xp = jnp.pad(x, ((p, p), (p, p), (0, 0)))

    def body(*refs):
        a_ref, b_ref = refs[0], refs[1]
        w_ref = refs[2]
        o_ref = refs[-1]
        xb = jnp.concatenate([a_ref[...], b_ref[...]], axis=0)
        acc = None
        for di in range(k):
            for dj in range(k):
                wt = w_ref[di * k + dj:di * k + dj + 1].reshape(1, 1, C)
                term = xb[di:di + Rh, dj:dj + W, :] * wt
                acc = term if acc is None else acc + term
        if residual:
            acc = acc + refs[3][...] + refs[4][...].reshape(1, 1, C)
        o_ref[...] = acc

    in_specs = [pl.BlockSpec((Rh, Wp, C), lambda i: (i, 0, 0)),
                pl.BlockSpec((2 * p, Wp, C),
                             lambda i: ((i + 1) * Rh // (2 * p), 0, 0)),
                pl.BlockSpec((k * k, C), lambda i: (0, 0))]
    args = [xp, xp, wf]
    if residual:
        in_specs += [pl.BlockSpec((Rh, W, C), lambda i: (i, 0, 0)),
                     pl.BlockSpec((1, C), lambda i: (0, 0))]
        args += [x, bias.reshape(1, C)]
    return _pcall(
        body,
        grid=(H // Rh,),
        in_specs=in_specs,
        out_specs=pl.BlockSpec((Rh, W, C), lambda i: (i, 0, 0)),
        out_shape=jax.ShapeDtypeStruct((H, W, C), _F32),
    )(*args)


def _ln_qv(x, g, b, wq, wv, bq, bv):
    HW, C = x.shape
    R = _rows(HW, C)

    def body(x_ref, g_ref, b_ref, wq_ref, wv_ref, bq_ref, bv_ref,
             xn_ref, q_ref, v_ref):
        x_ = x_ref[...]
        mu = jnp.mean(x_, axis=-1, keepdims=True)
        xc = x_ - mu
        var = jnp.mean(xc * xc, axis=-1, keepdims=True)
        xn = xc * lax.rsqrt(var + 1e-5) * g_ref[...] + b_ref[...]
        xn_ref[...] = xn
        q_ref[...] = jnp.dot(xn, wq_ref[...], preferred_element_type=_F32) + bq_ref[...]
        v_ref[...] = jnp.dot(xn, wv_ref[...], preferred_element_type=_F32) + bv_ref[...]

    full = lambda shape: pl.BlockSpec(shape, lambda i: (0,) * len(shape))
    row = pl.BlockSpec((R, C), lambda i: (i, 0))
    return _pcall(
        body,
        grid=(HW // R,),
        in_specs=[row, full((1, C)), full((1, C)), full((C, C)), full((C, C)),
                  full((1, C)), full((1, C))],
        out_specs=[row, row, row],
        out_shape=[jax.ShapeDtypeStruct((HW, C), _F32)] * 3,
    )(x, g.reshape(1, C), b.reshape(1, C), wq, wv,
      bq.reshape(1, C), bv.reshape(1, C))


def _route_proj(xp, xm, wq, wk, wv, bq, bk, bv, topk):
    Pm, C = xp.shape
    P = xm.shape[0]

    def body(xp_ref, xm_ref, wq_ref, wk_ref, wv_ref, bq_ref, bk_ref, bv_ref,
             kp_ref, vp_ref, idx_ref):
        xp_ = xp_ref[...]
        xm_ = xm_ref[...]
        kp_ref[...] = jnp.dot(xp_, wk_ref[...], preferred_element_type=_F32) + bk_ref[...]
        vp_ref[...] = jnp.dot(xp_, wv_ref[...], preferred_element_type=_F32) + bv_ref[...]
        qm = jnp.dot(xm_, wq_ref[...], preferred_element_type=_F32) + bq_ref[...]
        km = jnp.dot(xm_, wk_ref[...], preferred_element_type=_F32) + bk_ref[...]
        a = lax.dot_general(qm, km, (((1,), (1,)), ((), ())),
                            preferred_element_type=_F32)
        col = lax.broadcasted_iota(jnp.int32, (P, P), 1)
        cols = []
        for _ in range(topk):
            mx = jnp.max(a, axis=1, keepdims=True)
            am = jnp.min(jnp.where(a >= mx, col, P), axis=1, keepdims=True)
            cols.append(am)
            a = jnp.where(col == am, -jnp.inf, a)
        idx_ref[...] = jnp.concatenate(cols, axis=1)

    return _pcall(
        body,
        out_shape=[jax.ShapeDtypeStruct((Pm, C), _F32),
                   jax.ShapeDtypeStruct((Pm, C), _F32),
                   jax.ShapeDtypeStruct((P, topk), jnp.int32)],
    )(xp, xm, wq, wk, wv, bq.reshape(1, C), bk.reshape(1, C), bv.reshape(1, C))


def _attn(idxf, qw, kp, vp, nh, m, topk):
    P, hw, C = qw.shape
    c = C // nh
    scale = c ** -0.5
    G = 14 if P % 14 == 0 else (7 if P % 7 == 0 else 1)

    def body(idx_ref, qw_ref, kp_ref, vp_ref, ow_ref):
        i = pl.program_id(0)
        for g in range(G):
            q = qw_ref[g]
            ks_parts, vs_parts = [], []
            for t in range(topk):
                j = idx_ref[(i * G + g) * topk + t]
                ks_parts.append(kp_ref[pl.ds(j * m, m), :])
                vs_parts.append(vp_ref[pl.ds(j * m, m), :])
            ks = jnp.concatenate(ks_parts, 0) if topk > 1 else ks_parts[0]
            vs = jnp.concatenate(vs_parts, 0) if topk > 1 else vs_parts[0]
            outs = []
            for hh in range(nh):
                qh = q[:, hh * c:(hh + 1) * c] * scale
                kh = ks[:, hh * c:(hh + 1) * c]
                vh = vs[:, hh * c:(hh + 1) * c]
                logits = lax.dot_general(qh, kh, (((1,), (1,)), ((), ())),
                                         preferred_element_type=_F32)
                pa = jax.nn.softmax(logits, axis=-1)
                outs.append(jnp.dot(pa, vh, preferred_element_type=_F32))
            ow_ref[g] = jnp.concatenate(outs, 1) if nh > 1 else outs[0]

    grid_spec = pltpu.PrefetchScalarGridSpec(
        num_scalar_prefetch=1,
        grid=(P // G,),
        in_specs=[pl.BlockSpec((G, hw, C), lambda i, idx: (i, 0, 0)),
                  pl.BlockSpec((P * m, C), lambda i, idx: (0, 0)),
                  pl.BlockSpec((P * m, C), lambda i, idx: (0, 0))],
        out_specs=pl.BlockSpec((G, hw, C), lambda i, idx: (i, 0, 0)),
    )
    return _pcall(
        body,
        grid_spec=grid_spec,
        out_shape=jax.ShapeDtypeStruct((P, hw, C), _F32),
    )(idxf, qw, kp, vp)


def _wo_mlp(xres, aw, lepe, wo, wob, g, b, w1, b1, w2, b2):
    HW, C = xres.shape
    C2 = w1.shape[1]
    R = _rows(HW, C)

    def body(x_ref, a_ref, l_ref, wo_ref, wob_ref, g_ref, b_ref,
             w1_ref, b1_ref, w2_ref, b2_ref, o_ref):
        y = a_ref[...] + l_ref[...]
        x1 = x_ref[...] + jnp.dot(y, wo_ref[...],
                                  preferred_element_type=_F32) + wob_ref[...]
        mu = jnp.mean(x1, axis=-1, keepdims=True)
        xc = x1 - mu
        var = jnp.mean(xc * xc, axis=-1, keepdims=True)
        xn = xc * lax.rsqrt(var + 1e-5) * g_ref[...] + b_ref[...]
        h = jax.nn.gelu(jnp.dot(xn, w1_ref[...], preferred_element_type=_F32)
                        + b1_ref[...])
        o_ref[...] = x1 + jnp.dot(h, w2_ref[...],
                                  preferred_element_type=_F32) + b2_ref[...]

    full = lambda shape: pl.BlockSpec(shape, lambda i: (0,) * len(shape))
    row = pl.BlockSpec((R, C), lambda i: (i, 0))
    return _pcall(
        body,
        grid=(HW // R,),
        in_specs=[row, row, row, full((C, C)), full((1, C)), full((1, C)),
                  full((1, C)), full((C, C2)), full((1, C2)), full((C2, C)),
                  full((1, C))],
        out_specs=row,
        out_shape=jax.ShapeDtypeStruct((HW, C), _F32),
    )(xres, aw, lepe, wo, wob.reshape(1, C), g.reshape(1, C), b.reshape(1, C),
      w1, b1.reshape(1, C2), w2, b2.reshape(1, C))


def _block(x, p, n_win, nh, topk, kv_per_win):
    C = x.shape[-1]
    x = _dwconv(x[0], p['cpe_w'], 3, bias=p['cpe_b'], residual=True)[None]
    _, H, W, _ = x.shape
    h, w = H // n_win, W // n_win
    P = n_win * n_win
    m = kv_per_win * kv_per_win
    bh, bw = h // kv_per_win, w // kv_per_win
    HW = H * W

    wq, wk, wv = jnp.split(p['qkv_w'], 3, axis=1)
    bq, bk, bv = jnp.split(p['qkv_b'], 3)

    x2 = x[0]
    xn, q, v = _ln_qv(x2.reshape(HW, C), p['ln1_g'], p['ln1_b'],
                      wq, wv, bq, bv)

    xn3 = xn.reshape(H, W, C)
    xp = (xn3.reshape(n_win, kv_per_win, bh, n_win, kv_per_win, bw, C)
             .mean(axis=(2, 5))
             .transpose(0, 2, 1, 3, 4)
             .reshape(P * m, C))
    xm = xn3.reshape(n_win, h, n_win, w, C).mean(axis=(1, 3)).reshape(P, C)

    kp, vp, idx = _route_proj(xp, xm, wq, wk, wv, bq, bk, bv, topk)

    qw = _win(q.reshape(H, W, C), n_win, h, w, C)
    aw = _attn(idx.reshape(P * topk), qw, kp, vp, nh, m, topk)
    a_spat = _unwin(aw, n_win, h, w, C).reshape(HW, C)

    lepe = _dwconv(v.reshape(H, W, C), p['lepe_w'], 5).reshape(HW, C)

    out = _wo_mlp(x2.reshape(HW, C), a_spat, lepe, p['wo_w'], p['wo_b'],
                  p['ln2_g'], p['ln2_b'], p['mlp_w1'], p['mlp_b1'],
                  p['mlp_w2'], p['mlp_b2'])
    return out.reshape(1, H, W, C)


def kernel(img0, img1, warped_img0, warped_img1, mask, flow, c0_0, c0_1,
           c0_2, c0_3, c1_0, c1_1, c1_2, c1_3, mask_guide_0, mask_guide_1,
           mask_guide_2, params):
    p = params
    nhwc = lambda t: t.transpose(0, 2, 3, 1)
    inp = jnp.concatenate([img0, img1, mask, mask_guide_0, warped_img0,
                           warped_img1, c0_0, c1_0, flow], 1)
    x1 = _conv(nhwc(inp), p['patch_embed'])
    for bp in p['enc1']:
        x1 = _block(x1, bp, 14, 1, 6, 2)
    f1 = _conv(nhwc(jnp.concatenate([mask_guide_1, c0_1, c1_1], 1)),
               p['c_down1'])
    x2 = _pixel_unshuffle(_conv(x1, p['down1_2']), 2)
    x2 = jnp.concatenate([x2, f1], -1)
    for bp in p['enc2']:
        x2 = _block(x2, bp, 7, 2, 4, 1)
    f2 = _conv(nhwc(jnp.concatenate([mask_guide_2, c0_2, c1_2], 1)),
               p['c_down2'])
    x3 = _pixel_unshuffle(_conv(x2, p['down2_3']), 2)
    x3 = jnp.concatenate([x3, f2], -1)
    for bp in p['enc3']:
        x3 = _block(x3, bp, 7, 4, 4, 1)
    f3 = _conv(nhwc(jnp.concatenate([c0_3, c1_3], 1)), p['c_down3'])
    x4 = _pixel_unshuffle(_conv(x3, p['down3_4']), 2)
    x4 = jnp.concatenate([x4, f3], -1)
    for bp in p['latent']:
        x4 = _block(x4, bp, 7, 8, 4, 1)
    d3 = _pixel_shuffle(_conv(x4, p['up4_3']), 2)
    d3 = _conv(jnp.concatenate([d3, x3], -1), p['reduce3'])
    for bp in p['dec3']:
        d3 = _block(d3, bp, 7, 1, 4, 1)
    d2 = _pixel_shuffle(_conv(d3, p['up3_2']), 2)
    d2 = _conv(jnp.concatenate([d2, x2], -1), p['reduce2'])
    for bp in p['dec2']:
        d2 = _block(d2, bp, 7, 1, 4, 1)
    d1 = _pixel_shuffle(_conv(d2, p['up2_1']), 2)
    d1 = jnp.concatenate([d1, x1], -1)
    for bp in p['dec1']:
        d1 = _block(d1, bp, 7, 1, 4, 2)
    for bp in p['refine']:
        d1 = _block(d1, bp, 14, 1, 6, 2)
    return jax.nn.sigmoid(_conv(d1, p['out_w'])).transpose(0, 3, 1, 2)


# speedup vs baseline: 1.8403x; 1.0857x over previous
import functools
import numpy as np
import jax
import jax.numpy as jnp
from jax import lax
from jax.experimental import pallas as pl
from jax.experimental.pallas import tpu as pltpu

_F32 = jnp.float32


def _pcall(body, **kw):
    return pl.pallas_call(body, **kw)


def _rows(HW, C):
    for r in (2048, 1024, 784, 512, 448, 392, 256, 224, 112, 98, 56, 49, 28, 16, 8):
        if HW % r == 0 and r * C * 4 <= 1 << 20:
            return r
    return HW


def _conv(x, w, groups=1):
    return lax.conv_general_dilated(
        x, w.transpose(2, 3, 1, 0), (1, 1), 'SAME',
        dimension_numbers=('NHWC', 'HWIO', 'NHWC'),
        feature_group_count=groups)


def _pixel_unshuffle(x, r):
    B, H, W, C = x.shape
    x = x.reshape(B, H // r, r, W // r, r, C)
    return x.transpose(0, 1, 3, 5, 2, 4).reshape(B, H // r, W // r, C * r * r)


def _pixel_shuffle(x, r):
    B, H, W, C = x.shape
    x = x.reshape(B, H, W, C // (r * r), r, r)
    return x.transpose(0, 1, 4, 2, 5, 3).reshape(B, H * r, W * r, C // (r * r))


def _win(t, n_win, h, w, C):
    return (t.reshape(n_win, h, n_win, w, C)
             .transpose(0, 2, 1, 3, 4)
             .reshape(n_win * n_win, h * w, C))


def _unwin(t, n_win, h, w, C):
    return (t.reshape(n_win, n_win, h, w, C)
             .transpose(0, 2, 1, 3, 4)
             .reshape(n_win * h, n_win * w, C))


def _dwconv(x, w, k, bias=None, residual=False):
    H, W, C = x.shape
    p = k // 2
    Rh = 28
    nblk = H // Rh
    wf = w.transpose(2, 3, 1, 0).reshape(k * k, C)

    def body(*refs):
        m_ref, t_ref, b_ref, w_ref = refs[0], refs[1], refs[2], refs[3]
        o_ref = refs[-1]
        i = pl.program_id(0)
        zero = jnp.zeros((p, W, C), _F32)
        top = jnp.where(i == 0, zero, t_ref[p:, :, :])
        bot = jnp.where(i == nblk - 1, zero, b_ref[:p, :, :])
        xb = jnp.concatenate([top, m_ref[...], bot], axis=0)
        zcol = jnp.zeros((Rh + 2 * p, p, C), _F32)
        xb = jnp.concatenate([zcol, xb, zcol], axis=1)
        acc = None
        for di in range(k):
            for dj in range(k):
                wt = w_ref[di * k + dj:di * k + dj + 1].reshape(1, 1, C)
                term = xb[di:di + Rh, dj:dj + W, :] * wt
                acc = term if acc is None else acc + term
        if residual:
            acc = acc + m_ref[...] + refs[4][...].reshape(1, 1, C)
        o_ref[...] = acc

    hb = Rh // (2 * p)
    in_specs = [pl.BlockSpec((Rh, W, C), lambda i: (i, 0, 0)),
                pl.BlockSpec((2 * p, W, C),
                             lambda i: (jnp.maximum(i * hb - 1, 0), 0, 0)),
                pl.BlockSpec((2 * p, W, C),
                             lambda i: (jnp.minimum((i + 1) * hb, nblk * hb - 1), 0, 0)),
                pl.BlockSpec((k * k, C), lambda i: (0, 0))]
    args = [x, x, x, wf]
    if residual:
        in_specs += [pl.BlockSpec((1, C), lambda i: (0, 0))]
        args += [bias.reshape(1, C)]
    return _pcall(
        body,
        grid=(nblk,),
        in_specs=in_specs,
        out_specs=pl.BlockSpec((Rh, W, C), lambda i: (i, 0, 0)),
        out_shape=jax.ShapeDtypeStruct((H, W, C), _F32),
    )(*args)


def _ln_qv(x, g, b, wq, wv, bq, bv):
    HW, C = x.shape
    R = _rows(HW, C)

    def body(x_ref, g_ref, b_ref, wq_ref, wv_ref, bq_ref, bv_ref,
             xn_ref, q_ref, v_ref):
        x_ = x_ref[...]
        mu = jnp.mean(x_, axis=-1, keepdims=True)
        xc = x_ - mu
        var = jnp.mean(xc * xc, axis=-1, keepdims=True)
        xn = xc * lax.rsqrt(var + 1e-5) * g_ref[...] + b_ref[...]
        xn_ref[...] = xn
        q_ref[...] = jnp.dot(xn, wq_ref[...], preferred_element_type=_F32) + bq_ref[...]
        v_ref[...] = jnp.dot(xn, wv_ref[...], preferred_element_type=_F32) + bv_ref[...]

    full = lambda shape: pl.BlockSpec(shape, lambda i: (0,) * len(shape))
    row = pl.BlockSpec((R, C), lambda i: (i, 0))
    return _pcall(
        body,
        grid=(HW // R,),
        in_specs=[row, full((1, C)), full((1, C)), full((C, C)), full((C, C)),
                  full((1, C)), full((1, C))],
        out_specs=[row, row, row],
        out_shape=[jax.ShapeDtypeStruct((HW, C), _F32)] * 3,
    )(x, g.reshape(1, C), b.reshape(1, C), wq, wv,
      bq.reshape(1, C), bv.reshape(1, C))


def _route_proj(xp, xm, wq, wk, wv, bq, bk, bv, topk):
    Pm, C = xp.shape
    P = xm.shape[0]

    def body(xp_ref, xm_ref, wq_ref, wk_ref, wv_ref, bq_ref, bk_ref, bv_ref,
             kp_ref, vp_ref, idx_ref):
        xp_ = xp_ref[...]
        xm_ = xm_ref[...]
        kp_ref[...] = jnp.dot(xp_, wk_ref[...], preferred_element_type=_F32) + bk_ref[...]
        vp_ref[...] = jnp.dot(xp_, wv_ref[...], preferred_element_type=_F32) + bv_ref[...]
        qm = jnp.dot(xm_, wq_ref[...], preferred_element_type=_F32) + bq_ref[...]
        km = jnp.dot(xm_, wk_ref[...], preferred_element_type=_F32) + bk_ref[...]
        a = lax.dot_general(qm, km, (((1,), (1,)), ((), ())),
                            preferred_element_type=_F32)
        col = lax.broadcasted_iota(jnp.int32, (P, P), 1)
        cols = []
        for _ in range(topk):
            mx = jnp.max(a, axis=1, keepdims=True)
            am = jnp.min(jnp.where(a >= mx, col, P), axis=1, keepdims=True)
            cols.append(am)
            a = jnp.where(col == am, -jnp.inf, a)
        idx_ref[...] = jnp.concatenate(cols, axis=1)

    return _pcall(
        body,
        out_shape=[jax.ShapeDtypeStruct((Pm, C), _F32),
                   jax.ShapeDtypeStruct((Pm, C), _F32),
                   jax.ShapeDtypeStruct((P, topk), jnp.int32)],
    )(xp, xm, wq, wk, wv, bq.reshape(1, C), bk.reshape(1, C), bv.reshape(1, C))


def _attn(idxf, qw, kp, vp, nh, m, topk):
    P, hw, C = qw.shape
    c = C // nh
    scale = c ** -0.5
    G = 14 if P % 14 == 0 else (7 if P % 7 == 0 else 1)

    def body(idx_ref, qw_ref, kp_ref, vp_ref, ow_ref):
        i = pl.program_id(0)
        for g in range(G):
            q = qw_ref[g]
            ks_parts, vs_parts = [], []
            for t in range(topk):
                j = idx_ref[(i * G + g) * topk + t]
                ks_parts.append(kp_ref[pl.ds(j * m, m), :])
                vs_parts.append(vp_ref[pl.ds(j * m, m), :])
            ks = jnp.concatenate(ks_parts, 0) if topk > 1 else ks_parts[0]
            vs = jnp.concatenate(vs_parts, 0) if topk > 1 else vs_parts[0]
            outs = []
            for hh in range(nh):
                qh = q[:, hh * c:(hh + 1) * c] * scale
                kh = ks[:, hh * c:(hh + 1) * c]
                vh = vs[:, hh * c:(hh + 1) * c]
                logits = lax.dot_general(qh, kh, (((1,), (1,)), ((), ())),
                                         preferred_element_type=_F32)
                pa = jax.nn.softmax(logits, axis=-1)
                outs.append(jnp.dot(pa, vh, preferred_element_type=_F32))
            ow_ref[g] = jnp.concatenate(outs, 1) if nh > 1 else outs[0]

    grid_spec = pltpu.PrefetchScalarGridSpec(
        num_scalar_prefetch=1,
        grid=(P // G,),
        in_specs=[pl.BlockSpec((G, hw, C), lambda i, idx: (i, 0, 0)),
                  pl.BlockSpec((P * m, C), lambda i, idx: (0, 0)),
                  pl.BlockSpec((P * m, C), lambda i, idx: (0, 0))],
        out_specs=pl.BlockSpec((G, hw, C), lambda i, idx: (i, 0, 0)),
    )
    return _pcall(
        body,
        grid_spec=grid_spec,
        out_shape=jax.ShapeDtypeStruct((P, hw, C), _F32),
    )(idxf, qw, kp, vp)


def _wo_mlp(xres, aw, lepe, wo, wob, g, b, w1, b1, w2, b2):
    HW, C = xres.shape
    C2 = w1.shape[1]
    R = _rows(HW, C)

    def body(x_ref, a_ref, l_ref, wo_ref, wob_ref, g_ref, b_ref,
             w1_ref, b1_ref, w2_ref, b2_ref, o_ref):
        y = a_ref[...] + l_ref[...]
        x1 = x_ref[...] + jnp.dot(y, wo_ref[...],
                                  preferred_element_type=_F32) + wob_ref[...]
        mu = jnp.mean(x1, axis=-1, keepdims=True)
        xc = x1 - mu
        var = jnp.mean(xc * xc, axis=-1, keepdims=True)
        xn = xc * lax.rsqrt(var + 1e-5) * g_ref[...] + b_ref[...]
        h = jax.nn.gelu(jnp.dot(xn, w1_ref[...], preferred_element_type=_F32)
                        + b1_ref[...])
        o_ref[...] = x1 + jnp.dot(h, w2_ref[...],
                                  preferred_element_type=_F32) + b2_ref[...]

    full = lambda shape: pl.BlockSpec(shape, lambda i: (0,) * len(shape))
    row = pl.BlockSpec((R, C), lambda i: (i, 0))
    return _pcall(
        body,
        grid=(HW // R,),
        in_specs=[row, row, row, full((C, C)), full((1, C)), full((1, C)),
                  full((1, C)), full((C, C2)), full((1, C2)), full((C2, C)),
                  full((1, C))],
        out_specs=row,
        out_shape=jax.ShapeDtypeStruct((HW, C), _F32),
    )(xres, aw, lepe, wo, wob.reshape(1, C), g.reshape(1, C), b.reshape(1, C),
      w1, b1.reshape(1, C2), w2, b2.reshape(1, C))


def _block(x, p, n_win, nh, topk, kv_per_win):
    C = x.shape[-1]
    x = _dwconv(x[0], p['cpe_w'], 3, bias=p['cpe_b'], residual=True)[None]
    _, H, W, _ = x.shape
    h, w = H // n_win, W // n_win
    P = n_win * n_win
    m = kv_per_win * kv_per_win
    bh, bw = h // kv_per_win, w // kv_per_win
    HW = H * W

    wq, wk, wv = jnp.split(p['qkv_w'], 3, axis=1)
    bq, bk, bv = jnp.split(p['qkv_b'], 3)

    x2 = x[0]
    xn, q, v = _ln_qv(x2.reshape(HW, C), p['ln1_g'], p['ln1_b'],
                      wq, wv, bq, bv)

    xn3 = xn.reshape(H, W, C)
    xp = (xn3.reshape(n_win, kv_per_win, bh, n_win, kv_per_win, bw, C)
             .mean(axis=(2, 5))
             .transpose(0, 2, 1, 3, 4)
             .reshape(P * m, C))
    xm = xn3.reshape(n_win, h, n_win, w, C).mean(axis=(1, 3)).reshape(P, C)

    kp, vp, idx = _route_proj(xp, xm, wq, wk, wv, bq, bk, bv, topk)

    qw = _win(q.reshape(H, W, C), n_win, h, w, C)
    aw = _attn(idx.reshape(P * topk), qw, kp, vp, nh, m, topk)
    a_spat = _unwin(aw, n_win, h, w, C).reshape(HW, C)

    lepe = _dwconv(v.reshape(H, W, C), p['lepe_w'], 5).reshape(HW, C)

    out = _wo_mlp(x2.reshape(HW, C), a_spat, lepe, p['wo_w'], p['wo_b'],
                  p['ln2_g'], p['ln2_b'], p['mlp_w1'], p['mlp_b1'],
                  p['mlp_w2'], p['mlp_b2'])
    return out.reshape(1, H, W, C)


def kernel(img0, img1, warped_img0, warped_img1, mask, flow, c0_0, c0_1,
           c0_2, c0_3, c1_0, c1_1, c1_2, c1_3, mask_guide_0, mask_guide_1,
           mask_guide_2, params):
    p = params
    nhwc = lambda t: t.transpose(0, 2, 3, 1)
    inp = jnp.concatenate([img0, img1, mask, mask_guide_0, warped_img0,
                           warped_img1, c0_0, c1_0, flow], 1)
    x1 = _conv(nhwc(inp), p['patch_embed'])
    for bp in p['enc1']:
        x1 = _block(x1, bp, 14, 1, 6, 2)
    f1 = _conv(nhwc(jnp.concatenate([mask_guide_1, c0_1, c1_1], 1)),
               p['c_down1'])
    x2 = _pixel_unshuffle(_conv(x1, p['down1_2']), 2)
    x2 = jnp.concatenate([x2, f1], -1)
    for bp in p['enc2']:
        x2 = _block(x2, bp, 7, 2, 4, 1)
    f2 = _conv(nhwc(jnp.concatenate([mask_guide_2, c0_2, c1_2], 1)),
               p['c_down2'])
    x3 = _pixel_unshuffle(_conv(x2, p['down2_3']), 2)
    x3 = jnp.concatenate([x3, f2], -1)
    for bp in p['enc3']:
        x3 = _block(x3, bp, 7, 4, 4, 1)
    f3 = _conv(nhwc(jnp.concatenate([c0_3, c1_3], 1)), p['c_down3'])
    x4 = _pixel_unshuffle(_conv(x3, p['down3_4']), 2)
    x4 = jnp.concatenate([x4, f3], -1)
    for bp in p['latent']:
        x4 = _block(x4, bp, 7, 8, 4, 1)
    d3 = _pixel_shuffle(_conv(x4, p['up4_3']), 2)
    d3 = _conv(jnp.concatenate([d3, x3], -1), p['reduce3'])
    for bp in p['dec3']:
        d3 = _block(d3, bp, 7, 1, 4, 1)
    d2 = _pixel_shuffle(_conv(d3, p['up3_2']), 2)
    d2 = _conv(jnp.concatenate([d2, x2], -1), p['reduce2'])
    for bp in p['dec2']:
        d2 = _block(d2, bp, 7, 1, 4, 1)
    d1 = _pixel_shuffle(_conv(d2, p['up2_1']), 2)
    d1 = jnp.concatenate([d1, x1], -1)
    for bp in p['dec1']:
        d1 = _block(d1, bp, 7, 1, 4, 2)
    for bp in p['refine']:
        d1 = _block(d1, bp, 14, 1, 6, 2)
    return jax.nn.sigmoid(_conv(d1, p['out_w'])).transpose(0, 3, 1, 2)
